# baseline (speedup 1.0000x reference)
import functools

import jax
import jax.numpy as jnp
from jax import lax
from jax.experimental import pallas as pl
from jax.experimental.pallas import tpu as pltpu
from jax.experimental.pallas import tpu_sc as plsc

_NC = 2
_NS = 16
_NW = _NC * _NS
_L = 16

_NBINS = 1024
_CH = 16384


def _dark_body(x_ref, o_ref):
    xr = x_ref[0]
    o_ref[0] = jnp.minimum(jnp.minimum(xr[0], xr[1]), xr[2])


def _dark_kernel(x):
    b, c, h, w = x.shape
    rb = 64
    return pl.pallas_call(
        _dark_body,
        grid=(b, h // rb),
        in_specs=[pl.BlockSpec((1, c, rb, w), lambda i, j: (i, 0, j, 0))],
        out_specs=pl.BlockSpec((1, rb, w), lambda i, j: (i, j, 0)),
        out_shape=jax.ShapeDtypeStruct((b, h, w), x.dtype),
    )(x)


def _select_body(k, imsz, dark_hbm, xflat_hbm, out_hbm, buf, hist, merged,
                 sel_idx, gidx0, gidx1, gidx2, gbuf0, gbuf1, gbuf2, pout,
                 sem):
    wid = lax.axis_index("s") * _NC + lax.axis_index("c")
    lane = lax.iota(jnp.int32, _L)
    ones = jnp.ones((_L,), jnp.int32)
    nchunk = imsz // _CH
    ngroup = _CH // _L

    def zero_hist():
        def zh(i, _):
            hist[pl.ds(i * _L, _L)] = jnp.zeros((_L,), jnp.int32)
            return 0
        lax.fori_loop(0, (_NBINS * _L) // _L, zh, 0)

    def hist_pass(shift_hi, prefix, shift_bin):
        zero_hist()

        def chunk(ch, _):
            pltpu.sync_copy(dark_hbm.at[wid, pl.ds(ch * _CH, _CH)], buf)

            def group(g, _):
                v = buf[pl.ds(g * _L, _L)]
                bits = lax.bitcast_convert_type(v, jnp.int32)
                m = (bits >> shift_hi) == prefix
                binv = (bits >> shift_bin) & (_NBINS - 1)
                plsc.addupdate_scatter(
                    hist, [lane * _NBINS + binv], ones, mask=m
                )
                return 0

            lax.fori_loop(0, ngroup, group, 0)
            return 0

        lax.fori_loop(0, nchunk, chunk, 0)

    def scan_hist(rank):
        def merge(j, _):
            acc = jnp.zeros((_L,), jnp.int32)
            for ln in range(_L):
                acc = acc + hist[pl.ds(ln * _NBINS + j * _L, _L)]
            merged[pl.ds(j * _L, _L)] = acc
            return 0

        lax.fori_loop(0, _NBINS // _L, merge, 0)

        def scang(g2, carry):
            above, bfound, nabv, found = carry
            g = _NBINS // _L - 1 - g2
            v = merged[pl.ds(g * _L, _L)]
            cs = plsc.cumsum(v)
            total = lax.reduce_sum_p.bind(v, axes=(0,))
            s_excl = above + total - cs
            s_incl = s_excl + v
            cond = (s_excl < rank) & (s_incl >= rank)
            hit = jnp.any(cond)
            j_lane = plsc.all_reduce_ffs(cond)
            if j_lane.ndim:
                j_lane = lax.reduce_max_p.bind(j_lane, axes=(0,))
            nab_here = lax.reduce_sum_p.bind(
                jnp.where(cond, s_excl, 0), axes=(0,)
            )
            new = (~found) & hit
            return (
                above + total,
                jnp.where(new, g * _L + j_lane, bfound),
                jnp.where(new, nab_here, nabv),
                found | hit,
            )

        _, bfound, nabv, _ = lax.fori_loop(
            0, _NBINS // _L, scang,
            (jnp.int32(0), jnp.int32(0), jnp.int32(0), False),
        )
        return bfound, nabv

    hist_pass(30, jnp.int32(0), 20)
    b1, nab1 = scan_hist(k)
    rank2 = k - nab1
    hist_pass(20, b1, 10)
    b2, nab2 = scan_hist(rank2)
    pref20 = (b1 << 10) | b2
    rank3 = rank2 - nab2
    hist_pass(10, pref20, 0)
    b3, nab3 = scan_hist(rank3)
    tbits = (pref20 << 10) | b3
    r_t = rank3 - nab3

    def zs(i, _):
        sel_idx[pl.ds(i * _L, _L)] = jnp.zeros((_L,), jnp.int32)
        return 0

    lax.fori_loop(0, 18, zs, 0)

    def cchunk(ch, carry):
        pltpu.sync_copy(dark_hbm.at[wid, pl.ds(ch * _CH, _CH)], buf)

        def group(g, carry):
            spos, tcnt = carry
            v = buf[pl.ds(g * _L, _L)]
            bits = lax.bitcast_convert_type(v, jnp.int32)
            strict = bits > tbits
            tie = bits == tbits
            tie_i = tie.astype(jnp.int32)
            tie_excl = plsc.cumsum(tie_i) - tie_i + tcnt
            take = strict | (tie & (tie_excl < r_t))
            lin = ch * _CH + g * _L + lane
            plsc.store_compressed(sel_idx.at[pl.ds(spos, _L)], lin, mask=take)
            spos = spos + lax.reduce_sum_p.bind(
                take.astype(jnp.int32), axes=(0,)
            )
            tcnt = tcnt + lax.reduce_sum_p.bind(tie_i, axes=(0,))
            return spos, tcnt

        return lax.fori_loop(0, ngroup, group, carry)

    lax.fori_loop(0, nchunk, cchunk, (jnp.int32(0), jnp.int32(0)))

    base = wid * (3 * imsz)
    ngr = 17
    gidxs = [gidx0, gidx1, gidx2]
    gbufs = [gbuf0, gbuf1, gbuf2]

    for c3 in range(3):
        def gi(g, _, c3=c3):
            iv = sel_idx[pl.ds(g * _L, _L)]
            gidxs[c3][pl.ds(g * _L, _L)] = iv + (base + c3 * imsz)
            return 0

        lax.fori_loop(0, ngr + 1, gi, 0)

    cps = [
        pltpu.async_copy(xflat_hbm.at[gidxs[c3]], gbufs[c3], sem)
        for c3 in range(3)
    ]
    for cp in cps:
        cp.wait()

    for c3 in range(3):
        def rg(g, acc, c3=c3):
            m = (g * _L + lane) < k
            v = gbufs[c3][pl.ds(g * _L, _L)]
            return acc + jnp.where(m, v, 0.0)

        acc = lax.fori_loop(0, ngr, rg, jnp.zeros((_L,), jnp.float32))
        pout[pl.ds(c3 * _L, _L)] = acc

    pltpu.sync_copy(pout, out_hbm.at[wid])


def _select_kernel(dark, xflat, k):
    b, imsz = dark.shape
    mesh = plsc.VectorSubcoreMesh(core_axis_name="c", subcore_axis_name="s")
    fn = pl.kernel(
        functools.partial(_select_body, k, imsz),
        out_type=jax.ShapeDtypeStruct((b, 3 * _L), jnp.float32),
        mesh=mesh,
        compiler_params=pltpu.CompilerParams(needs_layout_passes=False),
        scratch_types=[
            pltpu.VMEM((_CH,), jnp.float32),
            pltpu.VMEM((_NBINS * _L,), jnp.int32),
            pltpu.VMEM((_NBINS,), jnp.int32),
            pltpu.VMEM((288,), jnp.int32),
            pltpu.VMEM((288,), jnp.int32),
            pltpu.VMEM((288,), jnp.int32),
            pltpu.VMEM((288,), jnp.int32),
            pltpu.VMEM((288,), jnp.float32),
            pltpu.VMEM((288,), jnp.float32),
            pltpu.VMEM((288,), jnp.float32),
            pltpu.VMEM((3 * _L,), jnp.float32),
            pltpu.SemaphoreType.DMA,
        ],
    )
    return fn(dark, xflat).reshape(b, 3, _L)


def _transform_body(k, x_ref, p_ref, o_ref):
    inv_k = 1.0 / k
    a = [
        lax.reduce_sum_p.bind(p_ref[0, c], axes=(0,)) * inv_k + 1e-6
        for c in range(3)
    ]
    x0 = x_ref[0, 0]
    x1 = x_ref[0, 1]
    x2 = x_ref[0, 2]
    m = jnp.minimum(
        jnp.minimum(x0 * (1.0 / a[0]), x1 * (1.0 / a[1])), x2 * (1.0 / a[2])
    )
    recip = 1.0 / jnp.maximum(1.0 - 0.75 * m, 0.1)
    o_ref[0, 0] = (x0 - a[0]) * recip + a[0]
    o_ref[0, 1] = (x1 - a[1]) * recip + a[1]
    o_ref[0, 2] = (x2 - a[2]) * recip + a[2]


def _transform_kernel(x, partials, k):
    b, c, h, w = x.shape
    rb = 64
    return pl.pallas_call(
        functools.partial(_transform_body, k),
        grid=(b, h // rb),
        in_specs=[
            pl.BlockSpec((1, c, rb, w), lambda i, j: (i, 0, j, 0)),
            pl.BlockSpec((1, 3, _L), lambda i, j: (i, 0, 0)),
        ],
        out_specs=pl.BlockSpec((1, c, rb, w), lambda i, j: (i, 0, j, 0)),
        out_shape=jax.ShapeDtypeStruct(x.shape, x.dtype),
    )(x, partials)


def kernel(x):
    b, c, h, w = x.shape
    imsz = h * w
    k = max(imsz // 1000, 1)
    dark = _dark_kernel(x)
    partials = _select_kernel(
        dark.reshape(b, imsz), x.reshape(b * c * imsz), k
    )
    return _transform_kernel(x, partials, k)


# speedup vs baseline: 1.0915x; 1.0915x over previous
import functools

import jax
import jax.numpy as jnp
from jax import lax
from jax.experimental import pallas as pl
from jax.experimental.pallas import tpu as pltpu
from jax.experimental.pallas import tpu_sc as plsc

_NC = 2
_NS = 16
_NW = _NC * _NS
_L = 16

_NBINS = 1024
_CH = 16384


def _dark_body(x_ref, o_ref):
    xr = x_ref[0]
    o_ref[0] = jnp.minimum(jnp.minimum(xr[0], xr[1]), xr[2])


def _dark_kernel(x):
    b, c, h, w = x.shape
    rb = 64
    return pl.pallas_call(
        _dark_body,
        grid=(b, h // rb),
        in_specs=[pl.BlockSpec((1, c, rb, w), lambda i, j: (i, 0, j, 0))],
        out_specs=pl.BlockSpec((1, rb, w), lambda i, j: (i, j, 0)),
        out_shape=jax.ShapeDtypeStruct((b, h, w), x.dtype),
    )(x)


def _select_body(k, imsz, dark_hbm, xflat_hbm, out_hbm, bufa, bufb, hist,
                 merged, sel_idx, gidx0, gidx1, gidx2, gbuf0, gbuf1, gbuf2,
                 pout, sema, semb, sem):
    wid = lax.axis_index("s") * _NC + lax.axis_index("c")
    lane = lax.iota(jnp.int32, _L)
    ones = jnp.ones((_L,), jnp.int32)
    lane_base = lane * _NBINS
    nchunk = imsz // _CH
    ngroup = _CH // _L

    def zero_hist():
        def zh(i, _):
            for u in range(8):
                hist[pl.ds((i * 8 + u) * _L, _L)] = jnp.zeros((_L,),
                                                              jnp.int32)
            return 0
        lax.fori_loop(0, (_NBINS * _L) // _L // 8, zh, 0)

    def chunk_at(ch):
        return dark_hbm.at[wid, pl.ds(ch * _CH, _CH)]

    def double_buffered(process, carry0):
        pltpu.async_copy(chunk_at(0), bufa, sema)

        def pair(j, carry):
            ch = j * 2
            pltpu.async_copy(chunk_at(ch + 1), bufb, semb)
            pltpu.make_async_copy(chunk_at(ch), bufa, sema).wait()
            carry = process(bufa, ch, carry)

            @pl.when(ch + 2 < nchunk)
            def _():
                pltpu.async_copy(chunk_at(ch + 2), bufa, sema)

            pltpu.make_async_copy(chunk_at(ch + 1), bufb, semb).wait()
            return process(bufb, ch + 1, carry)

        return lax.fori_loop(0, nchunk // 2, pair, carry0)

    def hist_pass(shift_hi, prefix, shift_bin):
        zero_hist()

        def process(buf, ch, carry):
            def group(g, _):
                for u in range(8):
                    v = buf[pl.ds((g * 8 + u) * _L, _L)]
                    bits = lax.bitcast_convert_type(v, jnp.int32)
                    binv = (bits >> shift_bin) & (_NBINS - 1)
                    if prefix is None:
                        m = None
                    else:
                        m = (bits >> shift_hi) == prefix
                    plsc.addupdate_scatter(
                        hist, [lane_base + binv], ones, mask=m
                    )
                return 0

            lax.fori_loop(0, ngroup // 8, group, 0)
            return carry

        double_buffered(process, 0)

    def scan_hist(rank):
        def merge(j, _):
            acc = jnp.zeros((_L,), jnp.int32)
            for ln in range(_L):
                acc = acc + hist[pl.ds(ln * _NBINS + j * _L, _L)]
            merged[pl.ds(j * _L, _L)] = acc
            return 0

        lax.fori_loop(0, _NBINS // _L, merge, 0)

        def scang(g2, carry):
            above, bfound, nabv, found = carry
            g = _NBINS // _L - 1 - g2
            v = merged[pl.ds(g * _L, _L)]
            cs = plsc.cumsum(v)
            total = lax.reduce_sum_p.bind(v, axes=(0,))
            s_excl = above + total - cs
            s_incl = s_excl + v
            cond = (s_excl < rank) & (s_incl >= rank)
            hit = jnp.any(cond)
            j_lane = plsc.all_reduce_ffs(cond)
            if j_lane.ndim:
                j_lane = lax.reduce_max_p.bind(j_lane, axes=(0,))
            nab_here = lax.reduce_sum_p.bind(
                jnp.where(cond, s_excl, 0), axes=(0,)
            )
            new = (~found) & hit
            return (
                above + total,
                jnp.where(new, g * _L + j_lane, bfound),
                jnp.where(new, nab_here, nabv),
                found | hit,
            )

        _, bfound, nabv, _ = lax.fori_loop(
            0, _NBINS // _L, scang,
            (jnp.int32(0), jnp.int32(0), jnp.int32(0), False),
        )
        return bfound, nabv

    hist_pass(30, None, 20)
    b1, nab1 = scan_hist(k)
    rank2 = k - nab1
    hist_pass(20, b1, 10)
    b2, nab2 = scan_hist(rank2)
    pref20 = (b1 << 10) | b2
    rank3 = rank2 - nab2
    hist_pass(10, pref20, 0)
    b3, nab3 = scan_hist(rank3)
    tbits = (pref20 << 10) | b3
    r_t = rank3 - nab3

    def zs(i, _):
        sel_idx[pl.ds(i * _L, _L)] = jnp.zeros((_L,), jnp.int32)
        return 0

    lax.fori_loop(0, 18, zs, 0)

    def cprocess(buf, ch, carry):
        def group(g, carry):
            for u in range(4):
                spos, tcnt = carry
                gg = g * 4 + u
                v = buf[pl.ds(gg * _L, _L)]
                bits = lax.bitcast_convert_type(v, jnp.int32)
                strict = bits > tbits
                tie = bits == tbits
                tie_i = tie.astype(jnp.int32)
                tie_excl = plsc.cumsum(tie_i) - tie_i + tcnt
                take = strict | (tie & (tie_excl < r_t))
                lin = ch * _CH + gg * _L + lane
                plsc.store_compressed(
                    sel_idx.at[pl.ds(spos, _L)], lin, mask=take
                )
                spos = spos + lax.reduce_sum_p.bind(
                    take.astype(jnp.int32), axes=(0,)
                )
                tcnt = tcnt + lax.reduce_sum_p.bind(tie_i, axes=(0,))
                carry = (spos, tcnt)
            return carry

        return lax.fori_loop(0, ngroup // 4, group, carry)

    double_buffered(cprocess, (jnp.int32(0), jnp.int32(0)))

    base = wid * (3 * imsz)
    ngr = 17
    gidxs = [gidx0, gidx1, gidx2]
    gbufs = [gbuf0, gbuf1, gbuf2]

    for c3 in range(3):
        def gi(g, _, c3=c3):
            iv = sel_idx[pl.ds(g * _L, _L)]
            gidxs[c3][pl.ds(g * _L, _L)] = iv + (base + c3 * imsz)
            return 0

        lax.fori_loop(0, ngr + 1, gi, 0)

    cps = [
        pltpu.async_copy(xflat_hbm.at[gidxs[c3]], gbufs[c3], sem)
        for c3 in range(3)
    ]
    for cp in cps:
        cp.wait()

    for c3 in range(3):
        def rg(g, acc, c3=c3):
            m = (g * _L + lane) < k
            v = gbufs[c3][pl.ds(g * _L, _L)]
            return acc + jnp.where(m, v, 0.0)

        acc = lax.fori_loop(0, ngr, rg, jnp.zeros((_L,), jnp.float32))
        pout[pl.ds(c3 * _L, _L)] = acc

    pltpu.sync_copy(pout, out_hbm.at[wid])


def _select_kernel(dark, xflat, k):
    b, imsz = dark.shape
    mesh = plsc.VectorSubcoreMesh(core_axis_name="c", subcore_axis_name="s")
    fn = pl.kernel(
        functools.partial(_select_body, k, imsz),
        out_type=jax.ShapeDtypeStruct((b, 3 * _L), jnp.float32),
        mesh=mesh,
        compiler_params=pltpu.CompilerParams(needs_layout_passes=False),
        scratch_types=[
            pltpu.VMEM((_CH,), jnp.float32),
            pltpu.VMEM((_CH,), jnp.float32),
            pltpu.VMEM((_NBINS * _L,), jnp.int32),
            pltpu.VMEM((_NBINS,), jnp.int32),
            pltpu.VMEM((288,), jnp.int32),
            pltpu.VMEM((288,), jnp.int32),
            pltpu.VMEM((288,), jnp.int32),
            pltpu.VMEM((288,), jnp.int32),
            pltpu.VMEM((288,), jnp.float32),
            pltpu.VMEM((288,), jnp.float32),
            pltpu.VMEM((288,), jnp.float32),
            pltpu.VMEM((3 * _L,), jnp.float32),
            pltpu.SemaphoreType.DMA,
            pltpu.SemaphoreType.DMA,
            pltpu.SemaphoreType.DMA,
        ],
    )
    return fn(dark, xflat).reshape(b, 3, _L)


def _transform_body(k, x_ref, p_ref, o_ref):
    inv_k = 1.0 / k
    a = [
        lax.reduce_sum_p.bind(p_ref[0, c], axes=(0,)) * inv_k + 1e-6
        for c in range(3)
    ]
    x0 = x_ref[0, 0]
    x1 = x_ref[0, 1]
    x2 = x_ref[0, 2]
    m = jnp.minimum(
        jnp.minimum(x0 * (1.0 / a[0]), x1 * (1.0 / a[1])), x2 * (1.0 / a[2])
    )
    recip = 1.0 / jnp.maximum(1.0 - 0.75 * m, 0.1)
    o_ref[0, 0] = (x0 - a[0]) * recip + a[0]
    o_ref[0, 1] = (x1 - a[1]) * recip + a[1]
    o_ref[0, 2] = (x2 - a[2]) * recip + a[2]


def _transform_kernel(x, partials, k):
    b, c, h, w = x.shape
    rb = 64
    return pl.pallas_call(
        functools.partial(_transform_body, k),
        grid=(b, h // rb),
        in_specs=[
            pl.BlockSpec((1, c, rb, w), lambda i, j: (i, 0, j, 0)),
            pl.BlockSpec((1, 3, _L), lambda i, j: (i, 0, 0)),
        ],
        out_specs=pl.BlockSpec((1, c, rb, w), lambda i, j: (i, 0, j, 0)),
        out_shape=jax.ShapeDtypeStruct(x.shape, x.dtype),
    )(x, partials)


def kernel(x):
    b, c, h, w = x.shape
    imsz = h * w
    k = max(imsz // 1000, 1)
    dark = _dark_kernel(x)
    partials = _select_kernel(
        dark.reshape(b, imsz), x.reshape(b * c * imsz), k
    )
    return _transform_kernel(x, partials, k)


# speedup vs baseline: 1.0947x; 1.0029x over previous
import functools

import jax
import jax.numpy as jnp
from jax import lax
from jax.experimental import pallas as pl
from jax.experimental.pallas import tpu as pltpu
from jax.experimental.pallas import tpu_sc as plsc

_NC = 2
_NS = 16
_NW = _NC * _NS
_L = 16

_NBINS = 1024
_CH = 16384


def _dark_body(x_ref, o_ref):
    xr = x_ref[0]
    o_ref[0] = jnp.minimum(jnp.minimum(xr[0], xr[1]), xr[2])


def _dark_kernel(x):
    b, c, h, w = x.shape
    rb = 64
    return pl.pallas_call(
        _dark_body,
        grid=(b, h // rb),
        in_specs=[pl.BlockSpec((1, c, rb, w), lambda i, j: (i, 0, j, 0))],
        out_specs=pl.BlockSpec((1, rb, w), lambda i, j: (i, j, 0)),
        out_shape=jax.ShapeDtypeStruct((b, h, w), x.dtype),
    )(x)


def _select_body(k, imsz, dark_hbm, xflat_hbm, out_hbm, bufa, bufb, hist,
                 merged, sel_idx, gidx0, gidx1, gidx2, gbuf0, gbuf1, gbuf2,
                 pout, sema, semb, sem):
    wid = lax.axis_index("s") * _NC + lax.axis_index("c")
    lane = lax.iota(jnp.int32, _L)
    ones = jnp.ones((_L,), jnp.int32)
    nchunk = imsz // _CH
    ngroup = _CH // _L

    def zero_hist():
        def zh(i, _):
            for u in range(8):
                hist[pl.ds((i * 8 + u) * _L, _L)] = jnp.zeros((_L,),
                                                              jnp.int32)
            return 0
        lax.fori_loop(0, (_NBINS * _L) // _L // 8, zh, 0)

    def chunk_at(ch):
        return dark_hbm.at[wid, pl.ds(ch * _CH, _CH)]

    def double_buffered(process, carry0):
        pltpu.async_copy(chunk_at(0), bufa, sema)

        def pair(j, carry):
            ch = j * 2
            pltpu.async_copy(chunk_at(ch + 1), bufb, semb)
            pltpu.make_async_copy(chunk_at(ch), bufa, sema).wait()
            carry = process(bufa, ch, carry)

            @pl.when(ch + 2 < nchunk)
            def _():
                pltpu.async_copy(chunk_at(ch + 2), bufa, sema)

            pltpu.make_async_copy(chunk_at(ch + 1), bufb, semb).wait()
            return process(bufb, ch + 1, carry)

        return lax.fori_loop(0, nchunk // 2, pair, carry0)

    def hist_pass(shift_hi, prefix, shift_bin):
        zero_hist()

        def process(buf, ch, carry):
            def group(g, _):
                for u in range(8):
                    v = buf[pl.ds((g * 8 + u) * _L, _L)]
                    bits = lax.bitcast_convert_type(v, jnp.int32)
                    binv = (bits >> shift_bin) & (_NBINS - 1)
                    if prefix is None:
                        m = None
                    else:
                        m = (bits >> shift_hi) == prefix
                    plsc.addupdate_scatter(
                        hist, [binv * _L + lane], ones, mask=m
                    )
                return 0

            lax.fori_loop(0, ngroup // 8, group, 0)
            return carry

        double_buffered(process, 0)

    def scan_hist(rank):
        def merge(j, _):
            bins16 = (j * _L + lane) * _L
            acc = jnp.zeros((_L,), jnp.int32)
            for t in range(_L):
                acc = acc + plsc.load_gather(hist, [bins16 + t])
            merged[pl.ds(j * _L, _L)] = acc
            return 0

        lax.fori_loop(0, _NBINS // _L, merge, 0)

        def scang(g2, carry):
            above, bfound, nabv, found = carry
            g = _NBINS // _L - 1 - g2
            v = merged[pl.ds(g * _L, _L)]
            cs = plsc.cumsum(v)
            total = lax.reduce_sum_p.bind(v, axes=(0,))
            s_excl = above + total - cs
            s_incl = s_excl + v
            cond = (s_excl < rank) & (s_incl >= rank)
            hit = jnp.any(cond)
            j_lane = plsc.all_reduce_ffs(cond)
            if j_lane.ndim:
                j_lane = lax.reduce_max_p.bind(j_lane, axes=(0,))
            nab_here = lax.reduce_sum_p.bind(
                jnp.where(cond, s_excl, 0), axes=(0,)
            )
            new = (~found) & hit
            return (
                above + total,
                jnp.where(new, g * _L + j_lane, bfound),
                jnp.where(new, nab_here, nabv),
                found | hit,
            )

        _, bfound, nabv, _ = lax.fori_loop(
            0, _NBINS // _L, scang,
            (jnp.int32(0), jnp.int32(0), jnp.int32(0), False),
        )
        return bfound, nabv

    hist_pass(30, None, 20)
    b1, nab1 = scan_hist(k)
    rank2 = k - nab1
    hist_pass(20, b1, 10)
    b2, nab2 = scan_hist(rank2)
    pref20 = (b1 << 10) | b2
    rank3 = rank2 - nab2
    hist_pass(10, pref20, 0)
    b3, nab3 = scan_hist(rank3)
    tbits = (pref20 << 10) | b3
    r_t = rank3 - nab3

    def zs(i, _):
        sel_idx[pl.ds(i * _L, _L)] = jnp.zeros((_L,), jnp.int32)
        return 0

    lax.fori_loop(0, 18, zs, 0)

    def cprocess(buf, ch, carry):
        def group(g, carry):
            for u in range(4):
                spos, tcnt = carry
                gg = g * 4 + u
                v = buf[pl.ds(gg * _L, _L)]
                bits = lax.bitcast_convert_type(v, jnp.int32)
                strict = bits > tbits
                tie = bits == tbits
                tie_i = tie.astype(jnp.int32)
                tie_excl = plsc.cumsum(tie_i) - tie_i + tcnt
                take = strict | (tie & (tie_excl < r_t))
                lin = ch * _CH + gg * _L + lane
                plsc.store_compressed(
                    sel_idx.at[pl.ds(spos, _L)], lin, mask=take
                )
                spos = spos + lax.reduce_sum_p.bind(
                    take.astype(jnp.int32), axes=(0,)
                )
                tcnt = tcnt + lax.reduce_sum_p.bind(tie_i, axes=(0,))
                carry = (spos, tcnt)
            return carry

        return lax.fori_loop(0, ngroup // 4, group, carry)

    double_buffered(cprocess, (jnp.int32(0), jnp.int32(0)))

    base = wid * (3 * imsz)
    ngr = 17
    gidxs = [gidx0, gidx1, gidx2]
    gbufs = [gbuf0, gbuf1, gbuf2]

    for c3 in range(3):
        def gi(g, _, c3=c3):
            iv = sel_idx[pl.ds(g * _L, _L)]
            gidxs[c3][pl.ds(g * _L, _L)] = iv + (base + c3 * imsz)
            return 0

        lax.fori_loop(0, ngr + 1, gi, 0)

    cps = [
        pltpu.async_copy(xflat_hbm.at[gidxs[c3]], gbufs[c3], sem)
        for c3 in range(3)
    ]
    for cp in cps:
        cp.wait()

    for c3 in range(3):
        def rg(g, acc, c3=c3):
            m = (g * _L + lane) < k
            v = gbufs[c3][pl.ds(g * _L, _L)]
            return acc + jnp.where(m, v, 0.0)

        acc = lax.fori_loop(0, ngr, rg, jnp.zeros((_L,), jnp.float32))
        pout[pl.ds(c3 * _L, _L)] = acc

    pltpu.sync_copy(pout, out_hbm.at[wid])


def _select_kernel(dark, xflat, k):
    b, imsz = dark.shape
    mesh = plsc.VectorSubcoreMesh(core_axis_name="c", subcore_axis_name="s")
    fn = pl.kernel(
        functools.partial(_select_body, k, imsz),
        out_type=jax.ShapeDtypeStruct((b, 3 * _L), jnp.float32),
        mesh=mesh,
        compiler_params=pltpu.CompilerParams(needs_layout_passes=False),
        scratch_types=[
            pltpu.VMEM((_CH,), jnp.float32),
            pltpu.VMEM((_CH,), jnp.float32),
            pltpu.VMEM((_NBINS * _L,), jnp.int32),
            pltpu.VMEM((_NBINS,), jnp.int32),
            pltpu.VMEM((288,), jnp.int32),
            pltpu.VMEM((288,), jnp.int32),
            pltpu.VMEM((288,), jnp.int32),
            pltpu.VMEM((288,), jnp.int32),
            pltpu.VMEM((288,), jnp.float32),
            pltpu.VMEM((288,), jnp.float32),
            pltpu.VMEM((288,), jnp.float32),
            pltpu.VMEM((3 * _L,), jnp.float32),
            pltpu.SemaphoreType.DMA,
            pltpu.SemaphoreType.DMA,
            pltpu.SemaphoreType.DMA,
        ],
    )
    return fn(dark, xflat).reshape(b, 3, _L)


def _transform_body(k, x_ref, p_ref, o_ref):
    inv_k = 1.0 / k
    a = [
        lax.reduce_sum_p.bind(p_ref[0, c], axes=(0,)) * inv_k + 1e-6
        for c in range(3)
    ]
    x0 = x_ref[0, 0]
    x1 = x_ref[0, 1]
    x2 = x_ref[0, 2]
    m = jnp.minimum(
        jnp.minimum(x0 * (1.0 / a[0]), x1 * (1.0 / a[1])), x2 * (1.0 / a[2])
    )
    recip = 1.0 / jnp.maximum(1.0 - 0.75 * m, 0.1)
    o_ref[0, 0] = (x0 - a[0]) * recip + a[0]
    o_ref[0, 1] = (x1 - a[1]) * recip + a[1]
    o_ref[0, 2] = (x2 - a[2]) * recip + a[2]


def _transform_kernel(x, partials, k):
    b, c, h, w = x.shape
    rb = 64
    return pl.pallas_call(
        functools.partial(_transform_body, k),
        grid=(b, h // rb),
        in_specs=[
            pl.BlockSpec((1, c, rb, w), lambda i, j: (i, 0, j, 0)),
            pl.BlockSpec((1, 3, _L), lambda i, j: (i, 0, 0)),
        ],
        out_specs=pl.BlockSpec((1, c, rb, w), lambda i, j: (i, 0, j, 0)),
        out_shape=jax.ShapeDtypeStruct(x.shape, x.dtype),
    )(x, partials)


def kernel(x):
    b, c, h, w = x.shape
    imsz = h * w
    k = max(imsz // 1000, 1)
    dark = _dark_kernel(x)
    partials = _select_kernel(
        dark.reshape(b, imsz), x.reshape(b * c * imsz), k
    )
    return _transform_kernel(x, partials, k)


# speedup vs baseline: 2.0002x; 1.8271x over previous
import functools

import jax
import jax.numpy as jnp
from jax import lax
from jax.experimental import pallas as pl
from jax.experimental.pallas import tpu as pltpu
from jax.experimental.pallas import tpu_sc as plsc

_NC = 2
_NS = 16
_NW = _NC * _NS
_L = 16

_NBINS = 1024
_CH = 16384


def _dark_body(x_ref, o_ref):
    xr = x_ref[0]
    o_ref[0] = jnp.minimum(jnp.minimum(xr[0], xr[1]), xr[2])


def _dark_kernel(x):
    b, c, h, w = x.shape
    rb = 64
    return pl.pallas_call(
        _dark_body,
        grid=(b, h // rb),
        in_specs=[pl.BlockSpec((1, c, rb, w), lambda i, j: (i, 0, j, 0))],
        out_specs=pl.BlockSpec((1, rb, w), lambda i, j: (i, j, 0)),
        out_shape=jax.ShapeDtypeStruct((b, h, w), x.dtype),
    )(x)


def _select_body(k, imsz, dark_hbm, xflat_hbm, out_hbm, bufa, bufb, hist,
                 merged, sel_idx, gidx0, gidx1, gidx2, gbuf0, gbuf1, gbuf2,
                 pout, sema, semb, sem):
    wid = lax.axis_index("s") * _NC + lax.axis_index("c")
    lane = lax.iota(jnp.int32, _L)
    ones = jnp.ones((_L,), jnp.int32)
    nchunk = imsz // _CH
    ngroup = _CH // _L

    def zero_hist():
        def zh(i, _):
            for u in range(8):
                hist[pl.ds((i * 8 + u) * _L, _L)] = jnp.zeros((_L,),
                                                              jnp.int32)
            return 0
        lax.fori_loop(0, (_NBINS * _L) // _L // 8, zh, 0)

    def chunk_at(ch):
        return dark_hbm.at[wid, pl.ds(ch * _CH, _CH)]

    def double_buffered(process, carry0):
        pltpu.async_copy(chunk_at(0), bufa, sema)

        def pair(j, carry):
            ch = j * 2
            pltpu.async_copy(chunk_at(ch + 1), bufb, semb)
            pltpu.make_async_copy(chunk_at(ch), bufa, sema).wait()
            carry = process(bufa, ch, carry)

            @pl.when(ch + 2 < nchunk)
            def _():
                pltpu.async_copy(chunk_at(ch + 2), bufa, sema)

            pltpu.make_async_copy(chunk_at(ch + 1), bufb, semb).wait()
            return process(bufb, ch + 1, carry)

        return lax.fori_loop(0, nchunk // 2, pair, carry0)

    def hist_pass(shift_hi, prefix, shift_bin):
        zero_hist()

        def process(buf, ch, carry):
            def group(g):
                v = buf[pl.ds(g * _L, _L)]
                bits = lax.bitcast_convert_type(v, jnp.int32)
                binv = (bits >> shift_bin) & (_NBINS - 1)
                if prefix is None:
                    m = None
                else:
                    m = (bits >> shift_hi) == prefix
                plsc.addupdate_scatter(
                    hist, [binv * _L + lane], ones, mask=m
                )

            plsc.parallel_loop(0, ngroup, unroll=8)(group)
            return carry

        double_buffered(process, 0)

    def scan_hist(rank):
        def merge(j, _):
            bins16 = (j * _L + lane) * _L
            acc = jnp.zeros((_L,), jnp.int32)
            for t in range(_L):
                acc = acc + plsc.load_gather(hist, [bins16 + t])
            merged[pl.ds(j * _L, _L)] = acc
            return 0

        lax.fori_loop(0, _NBINS // _L, merge, 0)

        def scang(g2, carry):
            above, bfound, nabv, found = carry
            g = _NBINS // _L - 1 - g2
            v = merged[pl.ds(g * _L, _L)]
            cs = plsc.cumsum(v)
            total = lax.reduce_sum_p.bind(v, axes=(0,))
            s_excl = above + total - cs
            s_incl = s_excl + v
            cond = (s_excl < rank) & (s_incl >= rank)
            hit = jnp.any(cond)
            j_lane = plsc.all_reduce_ffs(cond)
            if j_lane.ndim:
                j_lane = lax.reduce_max_p.bind(j_lane, axes=(0,))
            nab_here = lax.reduce_sum_p.bind(
                jnp.where(cond, s_excl, 0), axes=(0,)
            )
            new = (~found) & hit
            return (
                above + total,
                jnp.where(new, g * _L + j_lane, bfound),
                jnp.where(new, nab_here, nabv),
                found | hit,
            )

        _, bfound, nabv, _ = lax.fori_loop(
            0, _NBINS // _L, scang,
            (jnp.int32(0), jnp.int32(0), jnp.int32(0), False),
        )
        return bfound, nabv

    hist_pass(30, None, 20)
    b1, nab1 = scan_hist(k)
    rank2 = k - nab1
    hist_pass(20, b1, 10)
    b2, nab2 = scan_hist(rank2)
    pref20 = (b1 << 10) | b2
    rank3 = rank2 - nab2
    hist_pass(10, pref20, 0)
    b3, nab3 = scan_hist(rank3)
    tbits = (pref20 << 10) | b3
    r_t = rank3 - nab3

    def zs(i, _):
        sel_idx[pl.ds(i * _L, _L)] = jnp.zeros((_L,), jnp.int32)
        return 0

    lax.fori_loop(0, 18, zs, 0)

    def cprocess(buf, ch, carry):
        def group(g, carry):
            spos, tcnt = carry
            v = buf[pl.ds(g * _L, _L)]
            bits = lax.bitcast_convert_type(v, jnp.int32)
            strict = bits > tbits
            tie = bits == tbits
            tie_i = tie.astype(jnp.int32)
            tie_excl = plsc.cumsum(tie_i) - tie_i + tcnt
            take = strict | (tie & (tie_excl < r_t))
            take_i = take.astype(jnp.int32)
            pos = spos + plsc.cumsum(take_i) - take_i
            lin = ch * _CH + g * _L + lane
            plsc.store_scatter(sel_idx, [pos], lin, mask=take)
            spos = spos + plsc.all_reduce_population_count(take)
            tcnt = tcnt + plsc.all_reduce_population_count(tie)
            return spos, tcnt

        return plsc.parallel_loop(0, ngroup, unroll=4, carry=carry)(group)

    zvec = jnp.zeros((_L,), jnp.int32)
    double_buffered(cprocess, (zvec, zvec))

    base = wid * (3 * imsz)
    ngr = 17
    gidxs = [gidx0, gidx1, gidx2]
    gbufs = [gbuf0, gbuf1, gbuf2]

    for c3 in range(3):
        def gi(g, _, c3=c3):
            iv = sel_idx[pl.ds(g * _L, _L)]
            gidxs[c3][pl.ds(g * _L, _L)] = iv + (base + c3 * imsz)
            return 0

        lax.fori_loop(0, ngr + 1, gi, 0)

    cps = [
        pltpu.async_copy(xflat_hbm.at[gidxs[c3]], gbufs[c3], sem)
        for c3 in range(3)
    ]
    for cp in cps:
        cp.wait()

    for c3 in range(3):
        def rg(g, acc, c3=c3):
            m = (g * _L + lane) < k
            v = gbufs[c3][pl.ds(g * _L, _L)]
            return acc + jnp.where(m, v, 0.0)

        acc = lax.fori_loop(0, ngr, rg, jnp.zeros((_L,), jnp.float32))
        pout[pl.ds(c3 * _L, _L)] = acc

    pltpu.sync_copy(pout, out_hbm.at[wid])


def _select_kernel(dark, xflat, k):
    b, imsz = dark.shape
    mesh = plsc.VectorSubcoreMesh(core_axis_name="c", subcore_axis_name="s")
    fn = pl.kernel(
        functools.partial(_select_body, k, imsz),
        out_type=jax.ShapeDtypeStruct((b, 3 * _L), jnp.float32),
        mesh=mesh,
        compiler_params=pltpu.CompilerParams(needs_layout_passes=False),
        scratch_types=[
            pltpu.VMEM((_CH,), jnp.float32),
            pltpu.VMEM((_CH,), jnp.float32),
            pltpu.VMEM((_NBINS * _L,), jnp.int32),
            pltpu.VMEM((_NBINS,), jnp.int32),
            pltpu.VMEM((288,), jnp.int32),
            pltpu.VMEM((288,), jnp.int32),
            pltpu.VMEM((288,), jnp.int32),
            pltpu.VMEM((288,), jnp.int32),
            pltpu.VMEM((288,), jnp.float32),
            pltpu.VMEM((288,), jnp.float32),
            pltpu.VMEM((288,), jnp.float32),
            pltpu.VMEM((3 * _L,), jnp.float32),
            pltpu.SemaphoreType.DMA,
            pltpu.SemaphoreType.DMA,
            pltpu.SemaphoreType.DMA,
        ],
    )
    return fn(dark, xflat).reshape(b, 3, _L)


def _transform_body(k, x_ref, p_ref, o_ref):
    inv_k = 1.0 / k
    a = [
        lax.reduce_sum_p.bind(p_ref[0, c], axes=(0,)) * inv_k + 1e-6
        for c in range(3)
    ]
    x0 = x_ref[0, 0]
    x1 = x_ref[0, 1]
    x2 = x_ref[0, 2]
    m = jnp.minimum(
        jnp.minimum(x0 * (1.0 / a[0]), x1 * (1.0 / a[1])), x2 * (1.0 / a[2])
    )
    recip = 1.0 / jnp.maximum(1.0 - 0.75 * m, 0.1)
    o_ref[0, 0] = (x0 - a[0]) * recip + a[0]
    o_ref[0, 1] = (x1 - a[1]) * recip + a[1]
    o_ref[0, 2] = (x2 - a[2]) * recip + a[2]


def _transform_kernel(x, partials, k):
    b, c, h, w = x.shape
    rb = 64
    return pl.pallas_call(
        functools.partial(_transform_body, k),
        grid=(b, h // rb),
        in_specs=[
            pl.BlockSpec((1, c, rb, w), lambda i, j: (i, 0, j, 0)),
            pl.BlockSpec((1, 3, _L), lambda i, j: (i, 0, 0)),
        ],
        out_specs=pl.BlockSpec((1, c, rb, w), lambda i, j: (i, 0, j, 0)),
        out_shape=jax.ShapeDtypeStruct(x.shape, x.dtype),
    )(x, partials)


def kernel(x):
    b, c, h, w = x.shape
    imsz = h * w
    k = max(imsz // 1000, 1)
    dark = _dark_kernel(x)
    partials = _select_kernel(
        dark.reshape(b, imsz), x.reshape(b * c * imsz), k
    )
    return _transform_kernel(x, partials, k)


# speedup vs baseline: 2.5323x; 1.2661x over previous
import functools

import jax
import jax.numpy as jnp
from jax import lax
from jax.experimental import pallas as pl
from jax.experimental.pallas import tpu as pltpu
from jax.experimental.pallas import tpu_sc as plsc

_NC = 2
_NS = 16
_NW = _NC * _NS
_L = 16

_NBINS = 1024
_CH = 16384


def _dark_body(x_ref, o_ref):
    xr = x_ref[0]
    o_ref[0] = jnp.minimum(jnp.minimum(xr[0], xr[1]), xr[2])


def _dark_kernel(x):
    b, c, h, w = x.shape
    rb = 128
    return pl.pallas_call(
        _dark_body,
        grid=(b, h // rb),
        in_specs=[pl.BlockSpec((1, c, rb, w), lambda i, j: (i, 0, j, 0))],
        out_specs=pl.BlockSpec((1, rb, w), lambda i, j: (i, j, 0)),
        out_shape=jax.ShapeDtypeStruct((b, h, w), x.dtype),
    )(x)


def _select_body(k, imsz, dark_hbm, xflat_hbm, out_hbm, bufa, bufb, hist,
                 merged, sel_idx, gidx0, gidx1, gidx2, gbuf0, gbuf1, gbuf2,
                 pout, sema, semb, sem):
    wid = lax.axis_index("s") * _NC + lax.axis_index("c")
    lane = lax.iota(jnp.int32, _L)
    ones = jnp.ones((_L,), jnp.int32)
    nchunk = imsz // _CH
    ngroup = _CH // _L

    def zero_hist():
        def zh(i, _):
            for u in range(8):
                hist[pl.ds((i * 8 + u) * _L, _L)] = jnp.zeros((_L,),
                                                              jnp.int32)
            return 0
        lax.fori_loop(0, (_NBINS * _L) // _L // 8, zh, 0)

    def chunk_at(ch):
        return dark_hbm.at[wid, pl.ds(ch * _CH, _CH)]

    def double_buffered(process, carry0):
        pltpu.async_copy(chunk_at(0), bufa, sema)

        def pair(j, carry):
            ch = j * 2
            pltpu.async_copy(chunk_at(ch + 1), bufb, semb)
            pltpu.make_async_copy(chunk_at(ch), bufa, sema).wait()
            carry = process(bufa, ch, carry)

            @pl.when(ch + 2 < nchunk)
            def _():
                pltpu.async_copy(chunk_at(ch + 2), bufa, sema)

            pltpu.make_async_copy(chunk_at(ch + 1), bufb, semb).wait()
            return process(bufb, ch + 1, carry)

        return lax.fori_loop(0, nchunk // 2, pair, carry0)

    def hist_pass(shift_hi, prefix, shift_bin):
        zero_hist()

        def process(buf, ch, carry):
            def group(g):
                v = buf[pl.ds(g * _L, _L)]
                bits = lax.bitcast_convert_type(v, jnp.int32)
                binv = (bits >> shift_bin) & (_NBINS - 1)
                if prefix is None:
                    m = None
                else:
                    m = (bits >> shift_hi) == prefix
                plsc.addupdate_scatter(
                    hist, [binv * _L + lane], ones, mask=m
                )

            plsc.parallel_loop(0, ngroup, unroll=8)(group)
            return carry

        double_buffered(process, 0)

    def scan_hist(rank):
        def merge(j, _):
            bins16 = (j * _L + lane) * _L
            acc = jnp.zeros((_L,), jnp.int32)
            for t in range(_L):
                acc = acc + plsc.load_gather(hist, [bins16 + t])
            merged[pl.ds(j * _L, _L)] = acc
            return 0

        lax.fori_loop(0, _NBINS // _L, merge, 0)

        def scang(g2, carry):
            above, bfound, nabv, found = carry
            g = _NBINS // _L - 1 - g2
            v = merged[pl.ds(g * _L, _L)]
            cs = plsc.cumsum(v)
            total = lax.reduce_sum_p.bind(v, axes=(0,))
            s_excl = above + total - cs
            s_incl = s_excl + v
            cond = (s_excl < rank) & (s_incl >= rank)
            hit = jnp.any(cond)
            j_lane = plsc.all_reduce_ffs(cond)
            if j_lane.ndim:
                j_lane = lax.reduce_max_p.bind(j_lane, axes=(0,))
            nab_here = lax.reduce_sum_p.bind(
                jnp.where(cond, s_excl, 0), axes=(0,)
            )
            new = (~found) & hit
            return (
                above + total,
                jnp.where(new, g * _L + j_lane, bfound),
                jnp.where(new, nab_here, nabv),
                found | hit,
            )

        _, bfound, nabv, _ = lax.fori_loop(
            0, _NBINS // _L, scang,
            (jnp.int32(0), jnp.int32(0), jnp.int32(0), False),
        )
        return bfound, nabv

    hist_pass(30, None, 20)
    b1, nab1 = scan_hist(k)
    rank2 = k - nab1
    hist_pass(20, b1, 10)
    b2, nab2 = scan_hist(rank2)
    pref20 = (b1 << 10) | b2
    rank3 = rank2 - nab2
    hist_pass(10, pref20, 0)
    b3, nab3 = scan_hist(rank3)
    tbits = (pref20 << 10) | b3
    r_t = rank3 - nab3

    def zs(i, _):
        sel_idx[pl.ds(i * _L, _L)] = jnp.zeros((_L,), jnp.int32)
        return 0

    lax.fori_loop(0, 18, zs, 0)

    def cprocess(buf, ch, carry):
        def group(g, carry):
            spos, tcnt = carry
            v = buf[pl.ds(g * _L, _L)]
            bits = lax.bitcast_convert_type(v, jnp.int32)
            strict = bits > tbits
            tie = bits == tbits
            tie_i = tie.astype(jnp.int32)
            tie_excl = plsc.cumsum(tie_i) - tie_i + tcnt
            take = strict | (tie & (tie_excl < r_t))
            take_i = take.astype(jnp.int32)
            pos = spos + plsc.cumsum(take_i) - take_i
            lin = ch * _CH + g * _L + lane
            plsc.store_scatter(sel_idx, [pos], lin, mask=take)
            spos = spos + plsc.all_reduce_population_count(take)
            tcnt = tcnt + plsc.all_reduce_population_count(tie)
            return spos, tcnt

        return plsc.parallel_loop(0, ngroup, unroll=4, carry=carry)(group)

    zvec = jnp.zeros((_L,), jnp.int32)
    double_buffered(cprocess, (zvec, zvec))

    base = wid * (3 * imsz)
    ngr = 17
    gidxs = [gidx0, gidx1, gidx2]
    gbufs = [gbuf0, gbuf1, gbuf2]

    for c3 in range(3):
        def gi(g, _, c3=c3):
            iv = sel_idx[pl.ds(g * _L, _L)]
            gidxs[c3][pl.ds(g * _L, _L)] = iv + (base + c3 * imsz)
            return 0

        lax.fori_loop(0, ngr + 1, gi, 0)

    cps = [
        pltpu.async_copy(xflat_hbm.at[gidxs[c3]], gbufs[c3], sem)
        for c3 in range(3)
    ]
    for cp in cps:
        cp.wait()

    for c3 in range(3):
        def rg(g, acc, c3=c3):
            m = (g * _L + lane) < k
            v = gbufs[c3][pl.ds(g * _L, _L)]
            return acc + jnp.where(m, v, 0.0)

        acc = lax.fori_loop(0, ngr, rg, jnp.zeros((_L,), jnp.float32))
        pout[pl.ds(c3 * _L, _L)] = acc

    pltpu.sync_copy(pout, out_hbm.at[wid])


def _select_kernel(dark, xflat, k):
    b, imsz = dark.shape
    mesh = plsc.VectorSubcoreMesh(core_axis_name="c", subcore_axis_name="s")
    fn = pl.kernel(
        functools.partial(_select_body, k, imsz),
        out_type=jax.ShapeDtypeStruct((b, 3 * _L), jnp.float32),
        mesh=mesh,
        compiler_params=pltpu.CompilerParams(needs_layout_passes=False),
        scratch_types=[
            pltpu.VMEM((_CH,), jnp.float32),
            pltpu.VMEM((_CH,), jnp.float32),
            pltpu.VMEM((_NBINS * _L,), jnp.int32),
            pltpu.VMEM((_NBINS,), jnp.int32),
            pltpu.VMEM((288,), jnp.int32),
            pltpu.VMEM((288,), jnp.int32),
            pltpu.VMEM((288,), jnp.int32),
            pltpu.VMEM((288,), jnp.int32),
            pltpu.VMEM((288,), jnp.float32),
            pltpu.VMEM((288,), jnp.float32),
            pltpu.VMEM((288,), jnp.float32),
            pltpu.VMEM((3 * _L,), jnp.float32),
            pltpu.SemaphoreType.DMA,
            pltpu.SemaphoreType.DMA,
            pltpu.SemaphoreType.DMA,
        ],
    )
    return fn(dark, xflat).reshape(b, 3, _L)


def _transform_body(k, x_ref, p_ref, o_ref):
    inv_k = 1.0 / k
    a = [
        lax.reduce_sum_p.bind(p_ref[0, c], axes=(0,)) * inv_k + 1e-6
        for c in range(3)
    ]
    x0 = x_ref[0, 0]
    x1 = x_ref[0, 1]
    x2 = x_ref[0, 2]
    m = jnp.minimum(
        jnp.minimum(x0 * (1.0 / a[0]), x1 * (1.0 / a[1])), x2 * (1.0 / a[2])
    )
    recip = 1.0 / jnp.maximum(1.0 - 0.75 * m, 0.1)
    o_ref[0, 0] = (x0 - a[0]) * recip + a[0]
    o_ref[0, 1] = (x1 - a[1]) * recip + a[1]
    o_ref[0, 2] = (x2 - a[2]) * recip + a[2]


def _transform_kernel(x, partials, k):
    b, c, h, w = x.shape
    rb = 128
    return pl.pallas_call(
        functools.partial(_transform_body, k),
        grid=(b, h // rb),
        in_specs=[
            pl.BlockSpec((1, c, rb, w), lambda i, j: (i, 0, j, 0)),
            pl.BlockSpec((1, 3, _L), lambda i, j: (i, 0, 0)),
        ],
        out_specs=pl.BlockSpec((1, c, rb, w), lambda i, j: (i, 0, j, 0)),
        out_shape=jax.ShapeDtypeStruct(x.shape, x.dtype),
    )(x, partials)


def kernel(x):
    b, c, h, w = x.shape
    imsz = h * w
    k = max(imsz // 1000, 1)
    dark = _dark_kernel(x)
    partials = _select_kernel(
        dark.reshape(b, imsz), x.reshape(b * c * imsz), k
    )
    return _transform_kernel(x, partials, k)


# speedup vs baseline: 2.9199x; 1.1531x over previous
import functools

import jax
import jax.numpy as jnp
from jax import lax
from jax.experimental import pallas as pl
from jax.experimental.pallas import tpu as pltpu
from jax.experimental.pallas import tpu_sc as plsc

_NC = 2
_NS = 16
_NW = _NC * _NS
_L = 16

_NBINS = 1024
_CH = 16384


def _dark_body(x_ref, o_ref):
    xr = x_ref[0]
    o_ref[0] = jnp.minimum(jnp.minimum(xr[0], xr[1]), xr[2])


def _dark_kernel(x):
    b, c, h, w = x.shape
    rb = 256
    return pl.pallas_call(
        _dark_body,
        grid=(b, h // rb),
        in_specs=[pl.BlockSpec((1, c, rb, w), lambda i, j: (i, 0, j, 0))],
        out_specs=pl.BlockSpec((1, rb, w), lambda i, j: (i, j, 0)),
        out_shape=jax.ShapeDtypeStruct((b, h, w), x.dtype),
    )(x)


def _select_body(k, imsz, dark_hbm, xflat_hbm, out_hbm, bufa, bufb, hist,
                 merged, sel_idx, gidx0, gidx1, gidx2, gbuf0, gbuf1, gbuf2,
                 pout, sema, semb, sem):
    wid = lax.axis_index("s") * _NC + lax.axis_index("c")
    lane = lax.iota(jnp.int32, _L)
    ones = jnp.ones((_L,), jnp.int32)
    nchunk = imsz // _CH
    ngroup = _CH // _L

    def zero_hist():
        def zh(i, _):
            for u in range(8):
                hist[pl.ds((i * 8 + u) * _L, _L)] = jnp.zeros((_L,),
                                                              jnp.int32)
            return 0
        lax.fori_loop(0, (_NBINS * _L) // _L // 8, zh, 0)

    def chunk_at(ch):
        return dark_hbm.at[wid, pl.ds(ch * _CH, _CH)]

    def double_buffered(process, carry0):
        pltpu.async_copy(chunk_at(0), bufa, sema)

        def pair(j, carry):
            ch = j * 2
            pltpu.async_copy(chunk_at(ch + 1), bufb, semb)
            pltpu.make_async_copy(chunk_at(ch), bufa, sema).wait()
            carry = process(bufa, ch, carry)

            @pl.when(ch + 2 < nchunk)
            def _():
                pltpu.async_copy(chunk_at(ch + 2), bufa, sema)

            pltpu.make_async_copy(chunk_at(ch + 1), bufb, semb).wait()
            return process(bufb, ch + 1, carry)

        return lax.fori_loop(0, nchunk // 2, pair, carry0)

    def hist_pass(shift_hi, prefix, shift_bin):
        zero_hist()

        def process(buf, ch, carry):
            def group(g):
                v = buf[pl.ds(g * _L, _L)]
                bits = lax.bitcast_convert_type(v, jnp.int32)
                binv = (bits >> shift_bin) & (_NBINS - 1)
                if prefix is None:
                    m = None
                else:
                    m = (bits >> shift_hi) == prefix
                plsc.addupdate_scatter(
                    hist, [binv * _L + lane], ones, mask=m
                )

            plsc.parallel_loop(0, ngroup, unroll=8)(group)
            return carry

        double_buffered(process, 0)

    def scan_hist(rank):
        def merge(j, _):
            bins16 = (j * _L + lane) * _L
            acc = jnp.zeros((_L,), jnp.int32)
            for t in range(_L):
                acc = acc + plsc.load_gather(hist, [bins16 + t])
            merged[pl.ds(j * _L, _L)] = acc
            return 0

        lax.fori_loop(0, _NBINS // _L, merge, 0)

        def scang(g2, carry):
            above, bfound, nabv, found = carry
            g = _NBINS // _L - 1 - g2
            v = merged[pl.ds(g * _L, _L)]
            cs = plsc.cumsum(v)
            total = lax.reduce_sum_p.bind(v, axes=(0,))
            s_excl = above + total - cs
            s_incl = s_excl + v
            cond = (s_excl < rank) & (s_incl >= rank)
            hit = jnp.any(cond)
            j_lane = plsc.all_reduce_ffs(cond)
            if j_lane.ndim:
                j_lane = lax.reduce_max_p.bind(j_lane, axes=(0,))
            nab_here = lax.reduce_sum_p.bind(
                jnp.where(cond, s_excl, 0), axes=(0,)
            )
            new = (~found) & hit
            return (
                above + total,
                jnp.where(new, g * _L + j_lane, bfound),
                jnp.where(new, nab_here, nabv),
                found | hit,
            )

        _, bfound, nabv, _ = lax.fori_loop(
            0, _NBINS // _L, scang,
            (jnp.int32(0), jnp.int32(0), jnp.int32(0), False),
        )
        return bfound, nabv

    hist_pass(30, None, 20)
    b1, nab1 = scan_hist(k)
    rank2 = k - nab1
    hist_pass(20, b1, 10)
    b2, nab2 = scan_hist(rank2)
    pref20 = (b1 << 10) | b2
    rank3 = rank2 - nab2
    hist_pass(10, pref20, 0)
    b3, nab3 = scan_hist(rank3)
    tbits = (pref20 << 10) | b3
    r_t = rank3 - nab3

    def zs(i, _):
        sel_idx[pl.ds(i * _L, _L)] = jnp.zeros((_L,), jnp.int32)
        return 0

    lax.fori_loop(0, 18, zs, 0)

    def cprocess(buf, ch, carry):
        def group(g, carry):
            spos, tcnt = carry
            v = buf[pl.ds(g * _L, _L)]
            bits = lax.bitcast_convert_type(v, jnp.int32)
            strict = bits > tbits
            tie = bits == tbits
            tie_i = tie.astype(jnp.int32)
            tie_excl = plsc.cumsum(tie_i) - tie_i + tcnt
            take = strict | (tie & (tie_excl < r_t))
            take_i = take.astype(jnp.int32)
            pos = spos + plsc.cumsum(take_i) - take_i
            lin = ch * _CH + g * _L + lane
            plsc.store_scatter(sel_idx, [pos], lin, mask=take)
            spos = spos + plsc.all_reduce_population_count(take)
            tcnt = tcnt + plsc.all_reduce_population_count(tie)
            return spos, tcnt

        return plsc.parallel_loop(0, ngroup, unroll=4, carry=carry)(group)

    zvec = jnp.zeros((_L,), jnp.int32)
    double_buffered(cprocess, (zvec, zvec))

    base = wid * (3 * imsz)
    ngr = 17
    gidxs = [gidx0, gidx1, gidx2]
    gbufs = [gbuf0, gbuf1, gbuf2]

    for c3 in range(3):
        def gi(g, _, c3=c3):
            iv = sel_idx[pl.ds(g * _L, _L)]
            gidxs[c3][pl.ds(g * _L, _L)] = iv + (base + c3 * imsz)
            return 0

        lax.fori_loop(0, ngr + 1, gi, 0)

    cps = [
        pltpu.async_copy(xflat_hbm.at[gidxs[c3]], gbufs[c3], sem)
        for c3 in range(3)
    ]
    for cp in cps:
        cp.wait()

    for c3 in range(3):
        def rg(g, acc, c3=c3):
            m = (g * _L + lane) < k
            v = gbufs[c3][pl.ds(g * _L, _L)]
            return acc + jnp.where(m, v, 0.0)

        acc = lax.fori_loop(0, ngr, rg, jnp.zeros((_L,), jnp.float32))
        pout[pl.ds(c3 * _L, _L)] = acc

    pltpu.sync_copy(pout, out_hbm.at[wid])


def _select_kernel(dark, xflat, k):
    b, imsz = dark.shape
    mesh = plsc.VectorSubcoreMesh(core_axis_name="c", subcore_axis_name="s")
    fn = pl.kernel(
        functools.partial(_select_body, k, imsz),
        out_type=jax.ShapeDtypeStruct((b, 3 * _L), jnp.float32),
        mesh=mesh,
        compiler_params=pltpu.CompilerParams(needs_layout_passes=False),
        scratch_types=[
            pltpu.VMEM((_CH,), jnp.float32),
            pltpu.VMEM((_CH,), jnp.float32),
            pltpu.VMEM((_NBINS * _L,), jnp.int32),
            pltpu.VMEM((_NBINS,), jnp.int32),
            pltpu.VMEM((288,), jnp.int32),
            pltpu.VMEM((288,), jnp.int32),
            pltpu.VMEM((288,), jnp.int32),
            pltpu.VMEM((288,), jnp.int32),
            pltpu.VMEM((288,), jnp.float32),
            pltpu.VMEM((288,), jnp.float32),
            pltpu.VMEM((288,), jnp.float32),
            pltpu.VMEM((3 * _L,), jnp.float32),
            pltpu.SemaphoreType.DMA,
            pltpu.SemaphoreType.DMA,
            pltpu.SemaphoreType.DMA,
        ],
    )
    return fn(dark, xflat).reshape(b, 3, _L)


def _transform_body(k, x_ref, p_ref, o_ref):
    inv_k = 1.0 / k
    a = [
        lax.reduce_sum_p.bind(p_ref[0, c], axes=(0,)) * inv_k + 1e-6
        for c in range(3)
    ]
    x0 = x_ref[0, 0]
    x1 = x_ref[0, 1]
    x2 = x_ref[0, 2]
    m = jnp.minimum(
        jnp.minimum(x0 * (1.0 / a[0]), x1 * (1.0 / a[1])), x2 * (1.0 / a[2])
    )
    recip = 1.0 / jnp.maximum(1.0 - 0.75 * m, 0.1)
    o_ref[0, 0] = (x0 - a[0]) * recip + a[0]
    o_ref[0, 1] = (x1 - a[1]) * recip + a[1]
    o_ref[0, 2] = (x2 - a[2]) * recip + a[2]


def _transform_kernel(x, partials, k):
    b, c, h, w = x.shape
    rb = 256
    return pl.pallas_call(
        functools.partial(_transform_body, k),
        grid=(b, h // rb),
        in_specs=[
            pl.BlockSpec((1, c, rb, w), lambda i, j: (i, 0, j, 0)),
            pl.BlockSpec((1, 3, _L), lambda i, j: (i, 0, 0)),
        ],
        out_specs=pl.BlockSpec((1, c, rb, w), lambda i, j: (i, 0, j, 0)),
        out_shape=jax.ShapeDtypeStruct(x.shape, x.dtype),
    )(x, partials)


def kernel(x):
    b, c, h, w = x.shape
    imsz = h * w
    k = max(imsz // 1000, 1)
    dark = _dark_kernel(x)
    partials = _select_kernel(
        dark.reshape(b, imsz), x.reshape(b * c * imsz), k
    )
    return _transform_kernel(x, partials, k)


# speedup vs baseline: 3.1876x; 1.0917x over previous
import functools

import jax
import jax.numpy as jnp
from jax import lax
from jax.experimental import pallas as pl
from jax.experimental.pallas import tpu as pltpu
from jax.experimental.pallas import tpu_sc as plsc

_NC = 2
_NS = 16
_NW = _NC * _NS
_L = 16

_NBINS = 1024
_CH = 16384


def _dark_body(x_ref, o_ref):
    xr = x_ref[0]
    o_ref[0] = jnp.minimum(jnp.minimum(xr[0], xr[1]), xr[2])


def _dark_kernel(x):
    b, c, h, w = x.shape
    rb = 512
    return pl.pallas_call(
        _dark_body,
        grid=(b, h // rb),
        in_specs=[pl.BlockSpec((1, c, rb, w), lambda i, j: (i, 0, j, 0))],
        out_specs=pl.BlockSpec((1, rb, w), lambda i, j: (i, j, 0)),
        out_shape=jax.ShapeDtypeStruct((b, h, w), x.dtype),
    )(x)


def _select_body(k, imsz, dark_hbm, xflat_hbm, out_hbm, bufa, bufb, hist,
                 merged, sel_idx, gidx0, gidx1, gidx2, gbuf0, gbuf1, gbuf2,
                 pout, sema, semb, sem):
    wid = lax.axis_index("s") * _NC + lax.axis_index("c")
    lane = lax.iota(jnp.int32, _L)
    ones = jnp.ones((_L,), jnp.int32)
    nchunk = imsz // _CH
    ngroup = _CH // _L

    def zero_hist():
        def zh(i, _):
            for u in range(8):
                hist[pl.ds((i * 8 + u) * _L, _L)] = jnp.zeros((_L,),
                                                              jnp.int32)
            return 0
        lax.fori_loop(0, (_NBINS * _L) // _L // 8, zh, 0)

    def chunk_at(ch):
        return dark_hbm.at[wid, pl.ds(ch * _CH, _CH)]

    def double_buffered(process, carry0):
        pltpu.async_copy(chunk_at(0), bufa, sema)

        def pair(j, carry):
            ch = j * 2
            pltpu.async_copy(chunk_at(ch + 1), bufb, semb)
            pltpu.make_async_copy(chunk_at(ch), bufa, sema).wait()
            carry = process(bufa, ch, carry)

            @pl.when(ch + 2 < nchunk)
            def _():
                pltpu.async_copy(chunk_at(ch + 2), bufa, sema)

            pltpu.make_async_copy(chunk_at(ch + 1), bufb, semb).wait()
            return process(bufb, ch + 1, carry)

        return lax.fori_loop(0, nchunk // 2, pair, carry0)

    def hist_pass(shift_hi, prefix, shift_bin):
        zero_hist()

        def process(buf, ch, carry):
            def group(g):
                v = buf[pl.ds(g * _L, _L)]
                bits = lax.bitcast_convert_type(v, jnp.int32)
                binv = (bits >> shift_bin) & (_NBINS - 1)
                if prefix is None:
                    m = None
                else:
                    m = (bits >> shift_hi) == prefix
                plsc.addupdate_scatter(
                    hist, [binv * _L + lane], ones, mask=m
                )

            plsc.parallel_loop(0, ngroup, unroll=8)(group)
            return carry

        double_buffered(process, 0)

    def scan_hist(rank):
        def merge(j, _):
            bins16 = (j * _L + lane) * _L
            acc = jnp.zeros((_L,), jnp.int32)
            for t in range(_L):
                acc = acc + plsc.load_gather(hist, [bins16 + t])
            merged[pl.ds(j * _L, _L)] = acc
            return 0

        lax.fori_loop(0, _NBINS // _L, merge, 0)

        def scang(g2, carry):
            above, bfound, nabv, found = carry
            g = _NBINS // _L - 1 - g2
            v = merged[pl.ds(g * _L, _L)]
            cs = plsc.cumsum(v)
            total = lax.reduce_sum_p.bind(v, axes=(0,))
            s_excl = above + total - cs
            s_incl = s_excl + v
            cond = (s_excl < rank) & (s_incl >= rank)
            hit = jnp.any(cond)
            j_lane = plsc.all_reduce_ffs(cond)
            if j_lane.ndim:
                j_lane = lax.reduce_max_p.bind(j_lane, axes=(0,))
            nab_here = lax.reduce_sum_p.bind(
                jnp.where(cond, s_excl, 0), axes=(0,)
            )
            new = (~found) & hit
            return (
                above + total,
                jnp.where(new, g * _L + j_lane, bfound),
                jnp.where(new, nab_here, nabv),
                found | hit,
            )

        _, bfound, nabv, _ = lax.fori_loop(
            0, _NBINS // _L, scang,
            (jnp.int32(0), jnp.int32(0), jnp.int32(0), False),
        )
        return bfound, nabv

    hist_pass(30, None, 20)
    b1, nab1 = scan_hist(k)
    rank2 = k - nab1
    hist_pass(20, b1, 10)
    b2, nab2 = scan_hist(rank2)
    pref20 = (b1 << 10) | b2
    rank3 = rank2 - nab2
    hist_pass(10, pref20, 0)
    b3, nab3 = scan_hist(rank3)
    tbits = (pref20 << 10) | b3
    r_t = rank3 - nab3

    def zs(i, _):
        sel_idx[pl.ds(i * _L, _L)] = jnp.zeros((_L,), jnp.int32)
        return 0

    lax.fori_loop(0, 18, zs, 0)

    def cprocess(buf, ch, carry):
        def group(g, carry):
            spos, tcnt = carry
            v = buf[pl.ds(g * _L, _L)]
            bits = lax.bitcast_convert_type(v, jnp.int32)
            strict = bits > tbits
            tie = bits == tbits
            tie_i = tie.astype(jnp.int32)
            tie_excl = plsc.cumsum(tie_i) - tie_i + tcnt
            take = strict | (tie & (tie_excl < r_t))
            take_i = take.astype(jnp.int32)
            pos = spos + plsc.cumsum(take_i) - take_i
            lin = ch * _CH + g * _L + lane
            plsc.store_scatter(sel_idx, [pos], lin, mask=take)
            spos = spos + plsc.all_reduce_population_count(take)
            tcnt = tcnt + plsc.all_reduce_population_count(tie)
            return spos, tcnt

        return plsc.parallel_loop(0, ngroup, unroll=4, carry=carry)(group)

    zvec = jnp.zeros((_L,), jnp.int32)
    double_buffered(cprocess, (zvec, zvec))

    base = wid * (3 * imsz)
    ngr = 17
    gidxs = [gidx0, gidx1, gidx2]
    gbufs = [gbuf0, gbuf1, gbuf2]

    for c3 in range(3):
        def gi(g, _, c3=c3):
            iv = sel_idx[pl.ds(g * _L, _L)]
            gidxs[c3][pl.ds(g * _L, _L)] = iv + (base + c3 * imsz)
            return 0

        lax.fori_loop(0, ngr + 1, gi, 0)

    cps = [
        pltpu.async_copy(xflat_hbm.at[gidxs[c3]], gbufs[c3], sem)
        for c3 in range(3)
    ]
    for cp in cps:
        cp.wait()

    for c3 in range(3):
        def rg(g, acc, c3=c3):
            m = (g * _L + lane) < k
            v = gbufs[c3][pl.ds(g * _L, _L)]
            return acc + jnp.where(m, v, 0.0)

        acc = lax.fori_loop(0, ngr, rg, jnp.zeros((_L,), jnp.float32))
        pout[pl.ds(c3 * _L, _L)] = acc

    pltpu.sync_copy(pout, out_hbm.at[wid])


def _select_kernel(dark, xflat, k):
    b, imsz = dark.shape
    mesh = plsc.VectorSubcoreMesh(core_axis_name="c", subcore_axis_name="s")
    fn = pl.kernel(
        functools.partial(_select_body, k, imsz),
        out_type=jax.ShapeDtypeStruct((b, 3 * _L), jnp.float32),
        mesh=mesh,
        compiler_params=pltpu.CompilerParams(needs_layout_passes=False),
        scratch_types=[
            pltpu.VMEM((_CH,), jnp.float32),
            pltpu.VMEM((_CH,), jnp.float32),
            pltpu.VMEM((_NBINS * _L,), jnp.int32),
            pltpu.VMEM((_NBINS,), jnp.int32),
            pltpu.VMEM((288,), jnp.int32),
            pltpu.VMEM((288,), jnp.int32),
            pltpu.VMEM((288,), jnp.int32),
            pltpu.VMEM((288,), jnp.int32),
            pltpu.VMEM((288,), jnp.float32),
            pltpu.VMEM((288,), jnp.float32),
            pltpu.VMEM((288,), jnp.float32),
            pltpu.VMEM((3 * _L,), jnp.float32),
            pltpu.SemaphoreType.DMA,
            pltpu.SemaphoreType.DMA,
            pltpu.SemaphoreType.DMA,
        ],
    )
    return fn(dark, xflat).reshape(b, 3, _L)


def _transform_body(k, x_ref, p_ref, o_ref):
    inv_k = 1.0 / k
    a = [
        lax.reduce_sum_p.bind(p_ref[0, c], axes=(0,)) * inv_k + 1e-6
        for c in range(3)
    ]
    x0 = x_ref[0, 0]
    x1 = x_ref[0, 1]
    x2 = x_ref[0, 2]
    m = jnp.minimum(
        jnp.minimum(x0 * (1.0 / a[0]), x1 * (1.0 / a[1])), x2 * (1.0 / a[2])
    )
    recip = 1.0 / jnp.maximum(1.0 - 0.75 * m, 0.1)
    o_ref[0, 0] = (x0 - a[0]) * recip + a[0]
    o_ref[0, 1] = (x1 - a[1]) * recip + a[1]
    o_ref[0, 2] = (x2 - a[2]) * recip + a[2]


def _transform_kernel(x, partials, k):
    b, c, h, w = x.shape
    rb = 512
    return pl.pallas_call(
        functools.partial(_transform_body, k),
        grid=(b, h // rb),
        in_specs=[
            pl.BlockSpec((1, c, rb, w), lambda i, j: (i, 0, j, 0)),
            pl.BlockSpec((1, 3, _L), lambda i, j: (i, 0, 0)),
        ],
        out_specs=pl.BlockSpec((1, c, rb, w), lambda i, j: (i, 0, j, 0)),
        out_shape=jax.ShapeDtypeStruct(x.shape, x.dtype),
    )(x, partials)


def kernel(x):
    b, c, h, w = x.shape
    imsz = h * w
    k = max(imsz // 1000, 1)
    dark = _dark_kernel(x)
    partials = _select_kernel(
        dark.reshape(b, imsz), x.reshape(b * c * imsz), k
    )
    return _transform_kernel(x, partials, k)
